# baseline (speedup 1.0000x reference)
import jax
import jax.numpy as jnp
from jax import lax
from jax.experimental import pallas as pl
from jax.experimental.pallas import tpu as pltpu

_B, _IN, _OUT = 2048, 128, 128
_F32 = jnp.float32
_HI = lax.Precision.HIGHEST


def _dot(a, b, dims):
    return lax.dot_general(a, b, (dims, ((), ())),
                           preferred_element_type=_F32, precision=_HI)


def _loss_body(x_ref, t_ref, wT_ref, out_ref, st_ref):
    x = x_ref[...]
    t = t_ref[...]
    wT = wT_ref[...]
    ones_row = jnp.ones((1, _B), _F32)

    d = _dot(ones_row, x, ((1,), (0,)))

    lane_iota_row = lax.broadcasted_iota(jnp.int32, (1, _OUT), 1)

    def arow(o, carry):
        e = (lane_iota_row == o).astype(_F32)
        t_col = _dot(t, e, ((1,), (1,)))
        m = jnp.minimum(x, t_col)
        row = _dot(ones_row, m, ((1,), (0,)))
        st_ref[pl.ds(o, 1), :] = row
        return carry

    lax.fori_loop(0, _OUT, arow, 0)

    score_T = st_ref[...] / d
    score_T = jnp.where(jnp.isnan(score_T), 1.0, score_T)

    i_iota = lax.broadcasted_iota(jnp.int32, (_OUT, _IN), 1)
    lane_iota_in = lax.broadcasted_iota(jnp.int32, (1, _IN), 1)

    def crank(j, carry):
        rs, rw = carry
        ej = (lane_iota_in == j).astype(_F32)
        s_col = _dot(score_T, ej, ((1,), (1,)))
        w_col = _dot(wT, ej, ((1,), (1,)))
        tie = i_iota > j
        rs = rs + jnp.where(s_col > score_T, 1.0, 0.0) \
                + jnp.where((s_col == score_T) & tie, 1.0, 0.0)
        rw = rw + jnp.where(w_col > wT, 1.0, 0.0) \
                + jnp.where((w_col == wT) & tie, 1.0, 0.0)
        return rs, rw

    zeros = jnp.zeros((_OUT, _IN), _F32)
    rs, rw = lax.fori_loop(0, _IN, crank, (zeros, zeros))

    def dacc(r, acc):
        rf = r.astype(_F32)
        a_r = jnp.sum(jnp.where(rw == rf, wT, 0.0), axis=1, keepdims=True)
        b_r = jnp.sum(jnp.where(rs == rf, wT, 0.0), axis=1, keepdims=True)
        return acc + a_r * b_r

    ab = lax.fori_loop(0, _IN, dacc, jnp.zeros((_OUT, 1), _F32))

    w2 = jnp.sum(wT * wT)
    loss = (2.0 * w2 - 2.0 * jnp.sum(ab)) / (_IN * _OUT)
    out_ref[...] = jnp.broadcast_to(loss, (1, 1))


def kernel(x, y, t, w, base_w):
    del y, base_w
    wT = w.T
    out = pl.pallas_call(
        _loss_body,
        out_shape=jax.ShapeDtypeStruct((1, 1), _F32),
        scratch_shapes=[pltpu.VMEM((_OUT, _IN), _F32)],
    )(x, t, wT)
    return out[0, 0]
